# Initial kernel scaffold; baseline (speedup 1.0000x reference)
import jax
import jax.numpy as jnp
from jax.experimental import pallas as pl


def kernel(log_em, obs):
    raise NotImplementedError("write your pallas kernel here")


# speedup vs baseline: 2.3187x; 2.3187x over previous
import functools

import jax
import jax.numpy as jnp
from jax import lax
from jax.experimental import pallas as pl
from jax.experimental.pallas import tpu as pltpu
from jax.experimental.pallas import tpu_sc as plsc

_NC = 2
_NS = 16
_NW = _NC * _NS
_L = 16


def _sc_gather(log_em, obs):
    S, V = log_em.shape
    B = obs.shape[0]
    CHUNK = min(2048, B)
    U = 8
    n_rounds = (S + _NW - 1) // _NW

    mesh = plsc.VectorSubcoreMesh(
        core_axis_name="c", subcore_axis_name="s",
        num_cores=_NC, num_subcores=_NS)

    @functools.partial(
        pl.kernel, mesh=mesh,
        compiler_params=pltpu.CompilerParams(needs_layout_passes=False),
        out_type=jax.ShapeDtypeStruct((S, B), jnp.float32),
        scratch_types=[
            pltpu.VMEM((V,), jnp.float32),
            pltpu.VMEM((B,), jnp.int32),
            pltpu.VMEM((CHUNK,), jnp.float32),
        ],
    )
    def k(table_hbm, obs_hbm, out_hbm, row_v, idx_v, out_v):
        wid = lax.axis_index("s") * _NC + lax.axis_index("c")
        pltpu.sync_copy(obs_hbm, idx_v)

        def row_round(j, carry):
            r = j * _NW + wid

            @pl.when(r < S)
            def _():
                pltpu.sync_copy(table_hbm.at[r], row_v)

                def chunk_body(ci, c2):
                    base = ci * CHUNK

                    def g(i, c3):
                        off = i * (_L * U)
                        for u in range(U):
                            o2 = off + u * _L
                            idx = idx_v[pl.ds(base + o2, _L)]
                            out_v[pl.ds(o2, _L)] = plsc.load_gather(
                                row_v, [idx])
                        return c3

                    lax.fori_loop(0, CHUNK // (_L * U), g, 0)
                    pltpu.sync_copy(out_v, out_hbm.at[r, pl.ds(base, CHUNK)])
                    return c2

                lax.fori_loop(0, B // CHUNK, chunk_body, 0)

            return carry

        lax.fori_loop(0, n_rounds, row_round, 0)

    return k(log_em, obs)


def _tc_log_softmax(g):
    S, B = g.shape
    BLK = 2048

    def body(x_ref, o_ref):
        x = x_ref[...]
        m = jnp.max(x, axis=0, keepdims=True)
        e = jnp.exp(x - m)
        s = jnp.sum(e, axis=0, keepdims=True)
        o_ref[...] = (x - m) - jnp.log(s)

    return pl.pallas_call(
        body,
        grid=(B // BLK,),
        in_specs=[pl.BlockSpec((S, BLK), lambda i: (0, i))],
        out_specs=pl.BlockSpec((S, BLK), lambda i: (0, i)),
        out_shape=jax.ShapeDtypeStruct((S, B), jnp.float32),
    )(g)


def kernel(log_em, obs):
    g = _sc_gather(log_em, obs)
    return _tc_log_softmax(g)
